# Initial kernel scaffold; baseline (speedup 1.0000x reference)
import jax
import jax.numpy as jnp
from jax.experimental import pallas as pl


def kernel(input_ids, token_type_ids, position_ids, word_embeddings, position_embeddings, token_type_embeddings):
    raise NotImplementedError("write your pallas kernel here")


# speedup vs baseline: 6.8913x; 6.8913x over previous
import functools

import jax
import jax.numpy as jnp
from jax import lax
from jax.experimental import pallas as pl
from jax.experimental.pallas import tpu as pltpu
from jax.experimental.pallas import tpu_sc as plsc

H = 128
L = 16
NC, NS = 2, 16
NW = NC * NS
C = 128


def _emb_body(nchunk, wid_hbm, pid_hbm, tid_hbm, wtab_hbm, ftab_hbm, out_hbm,
              widx_v, pidx_v, fidx_v, wrows_v, frows_v, sem_g):
    w = lax.axis_index("s") * NC + lax.axis_index("c")
    npw = nchunk * C
    base = w * npw

    def chunk_body(g, _):
        off = base + g * C
        pltpu.sync_copy(wid_hbm.at[pl.ds(off, C)], widx_v)
        pltpu.sync_copy(pid_hbm.at[pl.ds(off, C)], pidx_v)
        pltpu.sync_copy(tid_hbm.at[pl.ds(off, C)], fidx_v)

        def fid_body(q, _):
            s = pl.ds(q * L, L)
            fidx_v[s] = pidx_v[s] * 2 + fidx_v[s]
            return 0
        lax.fori_loop(0, C // L, fid_body, 0)

        cp_w = pltpu.async_copy(wtab_hbm.at[widx_v], wrows_v, sem_g)
        cp_f = pltpu.async_copy(ftab_hbm.at[fidx_v], frows_v, sem_g)
        cp_w.wait()
        cp_f.wait()

        def add_body(i, _):
            for j in range(H // L):
                s = pl.ds(j * L, L)
                wrows_v[i, s] = wrows_v[i, s] + frows_v[i, s]
            return 0
        lax.fori_loop(0, C, add_body, 0)

        pltpu.sync_copy(wrows_v, out_hbm.at[pl.ds(off, C)])
        return 0

    lax.fori_loop(0, nchunk, chunk_body, 0)


def kernel(input_ids, token_type_ids, position_ids, word_embeddings,
           position_embeddings, token_type_embeddings):
    B, S = input_ids.shape
    n = B * S
    assert n % (NW * C) == 0
    nchunk = n // (NW * C)

    max_pos, h = position_embeddings.shape
    tvocab = token_type_embeddings.shape[0]
    assert h == H and tvocab == 2

    fused_tab = (position_embeddings[:, None, :]
                 + token_type_embeddings[None, :, :]).reshape(max_pos * tvocab, H)

    wid = input_ids.reshape(n).astype(jnp.int32)
    pid = position_ids.reshape(n).astype(jnp.int32)
    tid = token_type_ids.reshape(n).astype(jnp.int32)

    mesh = plsc.VectorSubcoreMesh(core_axis_name="c", subcore_axis_name="s",
                                  num_cores=NC, num_subcores=NS)
    run = pl.kernel(
        functools.partial(_emb_body, nchunk),
        out_type=jax.ShapeDtypeStruct((n, H), jnp.float32),
        mesh=mesh,
        scratch_types=[
            pltpu.VMEM((C,), jnp.int32),
            pltpu.VMEM((C,), jnp.int32),
            pltpu.VMEM((C,), jnp.int32),
            pltpu.VMEM((C, H), jnp.float32),
            pltpu.VMEM((C, H), jnp.float32),
            pltpu.SemaphoreType.DMA,
        ],
    )
    out = run(wid, pid, tid, word_embeddings, fused_tab)
    return out.reshape(B, S, H)
